# Initial kernel scaffold; baseline (speedup 1.0000x reference)
import jax
import jax.numpy as jnp
from jax.experimental import pallas as pl


def kernel(x, word_table, pos_table):
    raise NotImplementedError("write your pallas kernel here")


# speedup vs baseline: 11.1106x; 11.1106x over previous
import functools

import jax
import jax.numpy as jnp
from jax import lax
from jax.experimental import pallas as pl
from jax.experimental.pallas import tpu as pltpu
from jax.experimental.pallas import tpu_sc as plsc


def _make_kernel(B, V, E, S):
    info = plsc.get_sparse_core_info()
    NC, NS, L = info.num_cores, info.num_subcores, info.num_lanes
    NW = NC * NS
    CH = S
    assert B % (NW * CH) == 0 and E % L == 0
    BPW = B // NW
    NCH = BPW // CH

    mesh = plsc.VectorSubcoreMesh(core_axis_name="c", subcore_axis_name="s")

    def add_pos(rows, pos_v):
        @pl.loop(0, CH)
        def _(r):
            for c in range(E // L):
                sl = pl.ds(c * L, L)
                plsc.addupdate(rows.at[r, sl], pos_v[r, sl])

    def body(x_hbm, wt_hbm, pt_hbm, out_hbm,
             idx_a, idx_b, rows_a, rows_b, pos_v, sem_a, sem_b):
        wid = lax.axis_index("s") * NC + lax.axis_index("c")
        base = wid * BPW

        pltpu.sync_copy(pt_hbm, pos_v)

        def start_gather(chunk, idx_v, rows_v, sem):
            off = pl.multiple_of(base + chunk * CH, CH)
            pltpu.sync_copy(x_hbm.at[pl.ds(off, CH)], idx_v)
            pltpu.async_copy(wt_hbm.at[idx_v], rows_v, sem)

        def finish(chunk, idx_v, rows_v, sem):
            pltpu.make_async_copy(wt_hbm.at[idx_v], rows_v, sem).wait()
            add_pos(rows_v, pos_v)
            off = pl.multiple_of(base + chunk * CH, CH)
            pltpu.sync_copy(rows_v, out_hbm.at[pl.ds(off, CH)])

        start_gather(0, idx_a, rows_a, sem_a)
        start_gather(1, idx_b, rows_b, sem_b)

        @pl.loop(0, NCH, step=2)
        def _(g):
            finish(g, idx_a, rows_a, sem_a)

            @pl.when(g + 2 < NCH)
            def _():
                start_gather(g + 2, idx_a, rows_a, sem_a)

            finish(g + 1, idx_b, rows_b, sem_b)

            @pl.when(g + 3 < NCH)
            def _():
                start_gather(g + 3, idx_b, rows_b, sem_b)

    return pl.kernel(
        body,
        out_type=jax.ShapeDtypeStruct((B, E), jnp.float32),
        mesh=mesh,
        scratch_types=[
            pltpu.VMEM((CH,), jnp.int32),
            pltpu.VMEM((CH,), jnp.int32),
            pltpu.VMEM((CH, E), jnp.float32),
            pltpu.VMEM((CH, E), jnp.float32),
            pltpu.VMEM((S, E), jnp.float32),
            pltpu.SemaphoreType.DMA,
            pltpu.SemaphoreType.DMA,
        ],
    )


@jax.jit
def kernel(x, word_table, pos_table):
    N, S = x.shape
    V, E = word_table.shape
    flat = _make_kernel(N * S, V, E, S)(
        x.reshape(-1).astype(jnp.int32), word_table, pos_table
    )
    return flat.reshape(N, S, E)


# speedup vs baseline: 16.3275x; 1.4695x over previous
import jax
import jax.numpy as jnp
from jax import lax
from jax.experimental import pallas as pl
from jax.experimental.pallas import tpu as pltpu
from jax.experimental.pallas import tpu_sc as plsc

RING = 4
LOOKAHEAD = 2


def _make_kernel(B, V, E, S):
    info = plsc.get_sparse_core_info()
    NC, NS, L = info.num_cores, info.num_subcores, info.num_lanes
    NW = NC * NS
    CH = S
    assert B % (NW * CH) == 0 and E % L == 0
    BPW = B // NW
    NCH = BPW // CH

    mesh = plsc.VectorSubcoreMesh(core_axis_name="c", subcore_axis_name="s")

    def body(x_hbm, wt_hbm, pt_hbm, out_hbm, idx_all, pos_v, rows, gsems, wsems):
        wid = lax.axis_index("s") * NC + lax.axis_index("c")
        base = wid * BPW

        pltpu.sync_copy(x_hbm.at[pl.ds(wid * NCH, NCH), :], idx_all)
        pltpu.sync_copy(pt_hbm, pos_v)

        def gather(chunk, b):
            return pltpu.make_async_copy(
                wt_hbm.at[idx_all.at[chunk]], rows[b], gsems[b])

        def write(chunk, b):
            off = pl.multiple_of(base + chunk * CH, CH)
            return pltpu.make_async_copy(
                rows[b], out_hbm.at[pl.ds(off, CH)], wsems[b])

        for b in range(LOOKAHEAD):
            gather(b, b).start()

        @pl.loop(0, NCH, step=RING)
        def _(g):
            for b in range(RING):
                c = g + b
                gather(c, b).wait()

                @pl.loop(0, CH, unroll=4)
                def _(r):
                    for col in range(E // L):
                        sl = pl.ds(col * L, L)
                        plsc.addupdate(rows[b].at[r, sl], pos_v[r, sl])

                write(c, b).start()
                p = c + LOOKAHEAD

                @pl.when(p < NCH)
                def _():
                    pb = (b + LOOKAHEAD) % RING

                    @pl.when(p >= RING)
                    def _():
                        write(p - RING, pb).wait()

                    gather(p, pb).start()

        for b in range(RING):
            write(NCH - RING + b, b).wait()

    return pl.kernel(
        body,
        out_type=jax.ShapeDtypeStruct((B, E), jnp.float32),
        mesh=mesh,
        scratch_types=[
            pltpu.VMEM((NCH, CH), jnp.int32),
            pltpu.VMEM((S, E), jnp.float32),
            [pltpu.VMEM((CH, E), jnp.float32)] * RING,
            [pltpu.SemaphoreType.DMA] * RING,
            [pltpu.SemaphoreType.DMA] * RING,
        ],
    )


@jax.jit
def kernel(x, word_table, pos_table):
    N, S = x.shape
    V, E = word_table.shape
    flat = _make_kernel(N * S, V, E, S)(
        x.astype(jnp.int32), word_table, pos_table
    )
    return flat.reshape(N, S, E)


# speedup vs baseline: 17.4991x; 1.0718x over previous
import jax
import jax.numpy as jnp
from jax import lax
from jax.experimental import pallas as pl
from jax.experimental.pallas import tpu as pltpu
from jax.experimental.pallas import tpu_sc as plsc

RING = 4
LOOKAHEAD = 2


def _make_kernel(B, V, E, S):
    info = plsc.get_sparse_core_info()
    NC, NS, L = info.num_cores, info.num_subcores, info.num_lanes
    NW = NC * NS
    CH = S
    assert B % (NW * CH) == 0 and E % L == 0
    BPW = B // NW
    NCH = BPW // CH

    mesh = plsc.VectorSubcoreMesh(core_axis_name="c", subcore_axis_name="s")

    def body(x_hbm, wt_hbm, pt_hbm, out_hbm, idx_all, pos_v, rows, gsems, wsems):
        wid = lax.axis_index("s") * NC + lax.axis_index("c")
        base = wid * BPW

        pltpu.sync_copy(x_hbm.at[pl.ds(wid * NCH, NCH), :], idx_all)
        pltpu.sync_copy(pt_hbm, pos_v)

        def gather(chunk, b):
            return pltpu.make_async_copy(
                wt_hbm.at[idx_all.at[chunk]], rows[b], gsems[b])

        def write(chunk, b):
            off = pl.multiple_of(base + chunk * CH, CH)
            return pltpu.make_async_copy(
                rows[b], out_hbm.at[pl.ds(off, CH)], wsems[b])

        for b in range(LOOKAHEAD):
            gather(b, b).start()

        @pl.loop(0, NCH, step=RING)
        def _(g):
            for b in range(RING):
                c = g + b
                gather(c, b).wait()
                p = c + LOOKAHEAD

                @pl.when(p < NCH)
                def _():
                    pb = (b + LOOKAHEAD) % RING

                    @pl.when(p >= RING)
                    def _():
                        write(p - RING, pb).wait()

                    gather(p, pb).start()

                @pl.loop(0, CH, unroll=8)
                def _(r):
                    for col in range(E // L):
                        sl = pl.ds(col * L, L)
                        plsc.addupdate(rows[b].at[r, sl], pos_v[r, sl])

                write(c, b).start()

        for b in range(RING):
            write(NCH - RING + b, b).wait()

    return pl.kernel(
        body,
        out_type=jax.ShapeDtypeStruct((B, E), jnp.float32),
        mesh=mesh,
        scratch_types=[
            pltpu.VMEM((NCH, CH), jnp.int32),
            pltpu.VMEM((S, E), jnp.float32),
            [pltpu.VMEM((CH, E), jnp.float32)] * RING,
            [pltpu.SemaphoreType.DMA] * RING,
            [pltpu.SemaphoreType.DMA] * RING,
        ],
    )


@jax.jit
def kernel(x, word_table, pos_table):
    N, S = x.shape
    V, E = word_table.shape
    flat = _make_kernel(N * S, V, E, S)(
        x.astype(jnp.int32), word_table, pos_table
    )
    return flat.reshape(N, S, E)
